# baseline (speedup 1.0000x reference)
import jax
import jax.numpy as jnp
from jax.experimental import pallas as pl

_HIDDEN = 2048
_N_EXPERTS = 8
_RANK = 8
_ER = _N_EXPERTS * _RANK
_TBLK = 512


def _moe_lora_body(x_ref, w_ref, up_ref, o_ref):
    x = x_ref[...]

    zz = jnp.dot(x, w_ref[...], preferred_element_type=jnp.float32)
    z = zz[:, :_ER]
    logits = zz[:, _ER:]

    m1 = jnp.max(logits, axis=-1, keepdims=True)
    masked = jnp.where(logits >= m1, -1e30, logits)
    m2 = jnp.max(masked, axis=-1, keepdims=True)
    denom = 1.0 + jnp.exp(m2 - m1)
    scale = jnp.where(logits >= m2, jnp.exp(logits - m1) / denom, 0.0)

    col_e = jax.lax.broadcasted_iota(jnp.int32, (_N_EXPERTS, _ER), 1) // _RANK
    row_e = jax.lax.broadcasted_iota(jnp.int32, (_N_EXPERTS, _ER), 0)
    expand = (col_e == row_e).astype(jnp.float32)
    scale64 = jnp.dot(scale, expand, preferred_element_type=jnp.float32)

    c = z * scale64
    o_ref[...] = x + jnp.dot(c, up_ref[...], preferred_element_type=jnp.float32)


@jax.jit
def kernel(x, router_W, experts_down, experts_up):
    b, s, h = x.shape
    n_tok = b * s
    x2 = x.reshape(n_tok, h)
    dn_all = experts_down.transpose(1, 0, 2).reshape(h, _ER)
    w_cat = jnp.concatenate([dn_all, router_W.T], axis=1)
    up_all = experts_up.reshape(_ER, h)

    grid = (n_tok // _TBLK,)
    out = pl.pallas_call(
        _moe_lora_body,
        grid=grid,
        in_specs=[
            pl.BlockSpec((_TBLK, h), lambda i: (i, 0)),
            pl.BlockSpec((h, _ER + _N_EXPERTS), lambda i: (0, 0)),
            pl.BlockSpec((_ER, h), lambda i: (0, 0)),
        ],
        out_specs=pl.BlockSpec((_TBLK, h), lambda i: (i, 0)),
        out_shape=jax.ShapeDtypeStruct((n_tok, h), x.dtype),
    )(x2, w_cat, up_all)
    return out.reshape(b, s, h)


# speedup vs baseline: 1.0736x; 1.0736x over previous
import jax
import jax.numpy as jnp
from jax.experimental import pallas as pl

_HIDDEN = 2048
_N_EXPERTS = 8
_RANK = 8
_ER = _N_EXPERTS * _RANK
_TBLK = 1024


def _moe_lora_body(x_ref, w_ref, up_ref, o_ref):
    x = x_ref[...]

    zz = jnp.dot(x, w_ref[...], preferred_element_type=jnp.float32)
    z = zz[:, :_ER]
    logits = zz[:, _ER:]

    m1 = jnp.max(logits, axis=-1, keepdims=True)
    masked = jnp.where(logits >= m1, -1e30, logits)
    m2 = jnp.max(masked, axis=-1, keepdims=True)
    denom = 1.0 + jnp.exp(m2 - m1)
    scale = jnp.where(logits >= m2, jnp.exp(logits - m1) / denom, 0.0)

    col_e = jax.lax.broadcasted_iota(jnp.int32, (_N_EXPERTS, _ER), 1) // _RANK
    row_e = jax.lax.broadcasted_iota(jnp.int32, (_N_EXPERTS, _ER), 0)
    expand = (col_e == row_e).astype(jnp.float32)
    scale64 = jnp.dot(scale, expand, preferred_element_type=jnp.float32)

    c = z * scale64
    o_ref[...] = x + jnp.dot(c, up_ref[...], preferred_element_type=jnp.float32)


@jax.jit
def kernel(x, router_W, experts_down, experts_up):
    b, s, h = x.shape
    n_tok = b * s
    x2 = x.reshape(n_tok, h)
    dn_all = experts_down.transpose(1, 0, 2).reshape(h, _ER)
    w_cat = jnp.concatenate([dn_all, router_W.T], axis=1)
    up_all = experts_up.reshape(_ER, h)

    grid = (n_tok // _TBLK,)
    out = pl.pallas_call(
        _moe_lora_body,
        grid=grid,
        in_specs=[
            pl.BlockSpec((_TBLK, h), lambda i: (i, 0)),
            pl.BlockSpec((h, _ER + _N_EXPERTS), lambda i: (0, 0)),
            pl.BlockSpec((_ER, h), lambda i: (0, 0)),
        ],
        out_specs=pl.BlockSpec((_TBLK, h), lambda i: (i, 0)),
        out_shape=jax.ShapeDtypeStruct((n_tok, h), x.dtype),
    )(x2, w_cat, up_all)
    return out.reshape(b, s, h)
